# baseline (speedup 1.0000x reference)
import functools

import jax
import jax.numpy as jnp
from jax.experimental import pallas as pl
from jax.experimental.pallas import tpu as pltpu
from jax.experimental.pallas import tpu_sc as plsc

_B = 4096
_H = 200
_D = 32
_V = 1000000
_GW = 128
_NCHUNK = 4
_BB = 64


def _emit_gather(tab_hbm, i_hbm, o_hbm, n):
    def body(i_vmem, o_vmem):
        pltpu.sync_copy(tab_hbm.at[i_vmem.at[0]], o_vmem)

    pltpu.emit_pipeline(
        body,
        grid=(n // _GW,),
        in_specs=[pl.BlockSpec((1, _GW), lambda i: (0, i))],
        out_specs=[pl.BlockSpec((_GW, 128), lambda i: (i, 0))],
        core_axis_name=("c", "s"),
        dimension_semantics=(pltpu.PARALLEL,),
    )(i_hbm, o_hbm)


def _sc_gather2(item128, cidx, hidx):
    nb = cidx.shape[0]
    nh = hidx.shape[0]
    mesh = plsc.VectorSubcoreMesh(core_axis_name="c", subcore_axis_name="s")

    @functools.partial(
        pl.kernel,
        out_type=(jax.ShapeDtypeStruct((nb, 128), jnp.float32),
                  jax.ShapeDtypeStruct((nh, 128), jnp.float32)),
        mesh=mesh,
    )
    def k(it_hbm, ci_hbm, hi_hbm, co_hbm, ho_hbm):
        _emit_gather(it_hbm, ci_hbm, co_hbm, nb)
        _emit_gather(it_hbm, hi_hbm, ho_hbm, nh)

    return k(item128, cidx.reshape(1, nb), hidx.reshape(1, nh))


def _sc_gather_macro(table128, midx):
    n = midx.shape[0]
    mesh = plsc.VectorSubcoreMesh(core_axis_name="c", subcore_axis_name="s")

    @functools.partial(
        pl.kernel,
        out_type=jax.ShapeDtypeStruct((n, 128), table128.dtype),
        mesh=mesh,
    )
    def k(tab_hbm, i_hbm, o_hbm):
        _emit_gather(tab_hbm, i_hbm, o_hbm, n)

    return k(table128, midx.reshape(1, n))


def _quarter_select(macro, q):
    parts = [macro[..., k * _D:(k + 1) * _D] for k in range(4)]
    qb = jnp.broadcast_to(q, parts[0].shape)
    t01 = jnp.where(qb == 0, parts[0], parts[1])
    t23 = jnp.where(qb == 2, parts[2], parts[3])
    return jnp.where(qb < 2, t01, t23)


def _att_block(hist_ref, hid_ref, item_ref, iq_ref,
               wac_ref, wd_ref, wbc_ref, b1_ref, w2_ref,
               out_ref, s_scr):
    f32 = jnp.float32
    bf16 = jnp.bfloat16
    item = _quarter_select(item_ref[...], iq_ref[...])
    hq = (hid_ref[...] & 3)[:, :, None]
    hist = _quarter_select(hist_ref[...], hq)

    itemterm = (jnp.dot(item.astype(bf16), wbc_ref[...],
                        preferred_element_type=f32) + b1_ref[...])
    histf = hist.reshape(_BB * _H, _D)
    him = (hist * item[:, None, :]).reshape(_BB * _H, _D)
    ph = (jnp.dot(histf.astype(bf16), wac_ref[...], preferred_element_type=f32)
          + jnp.dot(him.astype(bf16), wd_ref[...], preferred_element_type=f32))
    h = jnp.maximum(ph.reshape(_BB, _H, 64) + itemterm[:, None, :], 0.0)

    s_scr[...] = jnp.sum(h * w2_ref[...][None, :, :], axis=-1)
    s2 = s_scr[...]
    m2 = jnp.max(s2, axis=1, keepdims=True)
    e2 = jnp.exp(s2 - m2)
    den2 = jnp.sum(e2, axis=1, keepdims=True)
    w2d = e2 * (1.0 / den2)
    wh = jnp.sum(hist * w2d[:, :, None], axis=1)

    out_ref[...] = jnp.concatenate([item, wh], axis=-1)


def _tc_attention(hist3, hid2, item_v, iq, wac, wd, wbc, b1, w2r):
    bc = hist3.shape[0]

    def wspec(shape):
        return pl.BlockSpec(shape, lambda i: (0,) * len(shape))

    return pl.pallas_call(
        _att_block,
        grid=(bc // _BB,),
        in_specs=[
            pl.BlockSpec((_BB, _H, 128), lambda i: (i, 0, 0)),
            pl.BlockSpec((_BB, _H), lambda i: (i, 0)),
            pl.BlockSpec((_BB, 128), lambda i: (i, 0)),
            pl.BlockSpec((_BB, 1), lambda i: (i, 0)),
            wspec((_D, 64)), wspec((_D, 64)), wspec((_D, 64)), wspec((1, 64)),
            wspec((1, 64)),
        ],
        out_specs=pl.BlockSpec((_BB, 2 * _D), lambda i: (i, 0)),
        out_shape=jax.ShapeDtypeStruct((bc, 2 * _D), jnp.float32),
        scratch_shapes=[pltpu.VMEM((_BB, _H), jnp.float32)],
    )(hist3, hid2, item_v, iq, wac, wd, wbc, b1, w2r)


_FB = 512


def _fin_block(user_ref, uq_ref, part_ref,
               mw1_ref, mb1_ref, mw2_ref, mb2_ref, mw3_ref, mb3_ref,
               out_ref):
    f32 = jnp.float32
    bf16 = jnp.bfloat16
    user = _quarter_select(user_ref[...], uq_ref[...])
    fi = jnp.concatenate([user, part_ref[...]], axis=-1)
    x1 = jnp.maximum(jnp.dot(fi.astype(bf16), mw1_ref[...],
                             preferred_element_type=f32) + mb1_ref[...], 0.0)
    x2 = jnp.maximum(jnp.dot(x1.astype(bf16), mw2_ref[...],
                             preferred_element_type=f32) + mb2_ref[...], 0.0)
    logit = jnp.sum(x2 * mw3_ref[...], axis=-1, keepdims=True) + mb3_ref[...]
    out_ref[...] = jax.nn.sigmoid(logit)


def _tc_final(user_v, uq, part, mw1, mb1, mw2, mb2, mw3r, mb3):
    def wspec(shape):
        return pl.BlockSpec(shape, lambda i: (0,) * len(shape))

    return pl.pallas_call(
        _fin_block,
        grid=(_B // _FB,),
        in_specs=[
            pl.BlockSpec((_FB, 128), lambda i: (i, 0)),
            pl.BlockSpec((_FB, 1), lambda i: (i, 0)),
            pl.BlockSpec((_FB, 2 * _D), lambda i: (i, 0)),
            wspec((3 * _D, 64)), wspec((1, 64)),
            wspec((64, _D)), wspec((1, _D)),
            wspec((1, _D)), wspec((1, 1)),
        ],
        out_specs=pl.BlockSpec((_FB, 1), lambda i: (i, 0)),
        out_shape=jax.ShapeDtypeStruct((_B, 1), jnp.float32),
    )(user_v, uq, part, mw1, mb1, mw2, mb2, mw3r, mb3)


def kernel(user_ids, candidate_items, history_items, user_emb, item_emb,
           att_w1, att_b1, att_w2, att_b2,
           mlp_w1, mlp_b1, mlp_w2, mlp_b2, mlp_w3, mlp_b3):
    bf16 = jnp.bfloat16
    wa, wb, wc, wdm = (att_w1[:_D], att_w1[_D:2 * _D],
                       att_w1[2 * _D:3 * _D], att_w1[3 * _D:])
    wac = (wa + wc).astype(bf16)
    wd = wdm.astype(bf16)
    wbc = (wb - wc).astype(bf16)
    b1 = att_b1.reshape(1, 64)
    w2r = att_w2.reshape(1, 64)
    mw1 = mlp_w1.astype(bf16)
    mb1 = mlp_b1.reshape(1, 64)
    mw2 = mlp_w2.astype(bf16)
    mb2 = mlp_b2.reshape(1, 32)
    mw3r = mlp_w3.reshape(1, 32)
    mb3 = mlp_b3.reshape(1, 1)

    def pack(table):
        tt = jnp.swapaxes(table, 0, 1)
        t3 = tt.reshape(_D, _V // 4, 4)
        return jnp.transpose(t3, (1, 2, 0)).reshape(_V // 4, 4 * _D)

    item128 = pack(item_emb)
    user128 = pack(user_emb)

    uid = user_ids.astype(jnp.int32)
    cid = candidate_items.astype(jnp.int32)
    hid = history_items.astype(jnp.int32)

    uq = (uid & 3).reshape(_B, 1)
    iq = (cid & 3).reshape(_B, 1)

    bc = _B // _NCHUNK
    item_v, hist0 = _sc_gather2(item128, cid >> 2,
                                (hid[:bc] >> 2).reshape(-1))

    hists = [hist0.reshape(bc, _H, 128)]
    for c in range(1, _NCHUNK):
        midx = (hid[c * bc:(c + 1) * bc] >> 2).reshape(-1)
        hists.append(_sc_gather_macro(item128, midx).reshape(bc, _H, 128))

    user_v = _sc_gather_macro(user128, uid >> 2)

    parts = []
    for c in range(_NCHUNK):
        sl = slice(c * bc, (c + 1) * bc)
        parts.append(_tc_attention(
            hists[c], hid[sl], item_v[sl], iq[sl], wac, wd, wbc, b1, w2r))
    part = jnp.concatenate(parts, axis=0)
    out = _tc_final(user_v, uq, part, mw1, mb1, mw2, mb2, mw3r, mb3)
    return out[:, 0]


# speedup vs baseline: 1.0238x; 1.0238x over previous
import functools

import jax
import jax.numpy as jnp
from jax.experimental import pallas as pl
from jax.experimental.pallas import tpu as pltpu
from jax.experimental.pallas import tpu_sc as plsc

_B = 4096
_H = 200
_D = 32
_V = 1000000
_GW = 128
_NCHUNK = 8
_BB = 64


def _emit_gather(tab_hbm, i_hbm, o_hbm, n):
    def body(i_vmem, o_vmem):
        pltpu.sync_copy(tab_hbm.at[i_vmem.at[0]], o_vmem)

    pltpu.emit_pipeline(
        body,
        grid=(n // _GW,),
        in_specs=[pl.BlockSpec((1, _GW), lambda i: (0, i))],
        out_specs=[pl.BlockSpec((_GW, 128), lambda i: (i, 0))],
        core_axis_name=("c", "s"),
        dimension_semantics=(pltpu.PARALLEL,),
    )(i_hbm, o_hbm)


def _sc_gather2(item128, cidx, hidx):
    nb = cidx.shape[0]
    nh = hidx.shape[0]
    mesh = plsc.VectorSubcoreMesh(core_axis_name="c", subcore_axis_name="s")

    @functools.partial(
        pl.kernel,
        out_type=(jax.ShapeDtypeStruct((nb, 128), jnp.float32),
                  jax.ShapeDtypeStruct((nh, 128), jnp.float32)),
        mesh=mesh,
    )
    def k(it_hbm, ci_hbm, hi_hbm, co_hbm, ho_hbm):
        _emit_gather(it_hbm, ci_hbm, co_hbm, nb)
        _emit_gather(it_hbm, hi_hbm, ho_hbm, nh)

    return k(item128, cidx.reshape(1, nb), hidx.reshape(1, nh))


def _sc_gather_macro(table128, midx):
    n = midx.shape[0]
    mesh = plsc.VectorSubcoreMesh(core_axis_name="c", subcore_axis_name="s")

    @functools.partial(
        pl.kernel,
        out_type=jax.ShapeDtypeStruct((n, 128), table128.dtype),
        mesh=mesh,
    )
    def k(tab_hbm, i_hbm, o_hbm):
        _emit_gather(tab_hbm, i_hbm, o_hbm, n)

    return k(table128, midx.reshape(1, n))


def _quarter_select(macro, q):
    parts = [macro[..., k * _D:(k + 1) * _D] for k in range(4)]
    qb = jnp.broadcast_to(q, parts[0].shape)
    t01 = jnp.where(qb == 0, parts[0], parts[1])
    t23 = jnp.where(qb == 2, parts[2], parts[3])
    return jnp.where(qb < 2, t01, t23)


def _att_block(hist_ref, hid_ref, item_ref, iq_ref,
               wac_ref, wd_ref, wbc_ref, b1_ref, w2_ref,
               out_ref, s_scr):
    f32 = jnp.float32
    bf16 = jnp.bfloat16
    item = _quarter_select(item_ref[...], iq_ref[...])
    hq = (hid_ref[...] & 3)[:, :, None]
    hist = _quarter_select(hist_ref[...], hq)

    itemterm = (jnp.dot(item.astype(bf16), wbc_ref[...],
                        preferred_element_type=f32) + b1_ref[...])
    histf = hist.reshape(_BB * _H, _D)
    him = (hist * item[:, None, :]).reshape(_BB * _H, _D)
    ph = (jnp.dot(histf.astype(bf16), wac_ref[...], preferred_element_type=f32)
          + jnp.dot(him.astype(bf16), wd_ref[...], preferred_element_type=f32))
    h = jnp.maximum(ph.reshape(_BB, _H, 64) + itemterm[:, None, :], 0.0)

    s_scr[...] = jnp.sum(h * w2_ref[...][None, :, :], axis=-1)
    s2 = s_scr[...]
    m2 = jnp.max(s2, axis=1, keepdims=True)
    e2 = jnp.exp(s2 - m2)
    den2 = jnp.sum(e2, axis=1, keepdims=True)
    w2d = e2 * (1.0 / den2)
    wh = jnp.sum(hist * w2d[:, :, None], axis=1)

    out_ref[...] = jnp.concatenate([item, wh], axis=-1)


def _tc_attention(hist3, hid2, item_v, iq, wac, wd, wbc, b1, w2r):
    bc = hist3.shape[0]

    def wspec(shape):
        return pl.BlockSpec(shape, lambda i: (0,) * len(shape))

    return pl.pallas_call(
        _att_block,
        grid=(bc // _BB,),
        in_specs=[
            pl.BlockSpec((_BB, _H, 128), lambda i: (i, 0, 0)),
            pl.BlockSpec((_BB, _H), lambda i: (i, 0)),
            pl.BlockSpec((_BB, 128), lambda i: (i, 0)),
            pl.BlockSpec((_BB, 1), lambda i: (i, 0)),
            wspec((_D, 64)), wspec((_D, 64)), wspec((_D, 64)), wspec((1, 64)),
            wspec((1, 64)),
        ],
        out_specs=pl.BlockSpec((_BB, 2 * _D), lambda i: (i, 0)),
        out_shape=jax.ShapeDtypeStruct((bc, 2 * _D), jnp.float32),
        scratch_shapes=[pltpu.VMEM((_BB, _H), jnp.float32)],
    )(hist3, hid2, item_v, iq, wac, wd, wbc, b1, w2r)


_FB = 512


def _fin_block(user_ref, uq_ref, part_ref,
               mw1_ref, mb1_ref, mw2_ref, mb2_ref, mw3_ref, mb3_ref,
               out_ref):
    f32 = jnp.float32
    bf16 = jnp.bfloat16
    user = _quarter_select(user_ref[...], uq_ref[...])
    fi = jnp.concatenate([user, part_ref[...]], axis=-1)
    x1 = jnp.maximum(jnp.dot(fi.astype(bf16), mw1_ref[...],
                             preferred_element_type=f32) + mb1_ref[...], 0.0)
    x2 = jnp.maximum(jnp.dot(x1.astype(bf16), mw2_ref[...],
                             preferred_element_type=f32) + mb2_ref[...], 0.0)
    logit = jnp.sum(x2 * mw3_ref[...], axis=-1, keepdims=True) + mb3_ref[...]
    out_ref[...] = jax.nn.sigmoid(logit)


def _tc_final(user_v, uq, part, mw1, mb1, mw2, mb2, mw3r, mb3):
    def wspec(shape):
        return pl.BlockSpec(shape, lambda i: (0,) * len(shape))

    return pl.pallas_call(
        _fin_block,
        grid=(_B // _FB,),
        in_specs=[
            pl.BlockSpec((_FB, 128), lambda i: (i, 0)),
            pl.BlockSpec((_FB, 1), lambda i: (i, 0)),
            pl.BlockSpec((_FB, 2 * _D), lambda i: (i, 0)),
            wspec((3 * _D, 64)), wspec((1, 64)),
            wspec((64, _D)), wspec((1, _D)),
            wspec((1, _D)), wspec((1, 1)),
        ],
        out_specs=pl.BlockSpec((_FB, 1), lambda i: (i, 0)),
        out_shape=jax.ShapeDtypeStruct((_B, 1), jnp.float32),
    )(user_v, uq, part, mw1, mb1, mw2, mb2, mw3r, mb3)


def kernel(user_ids, candidate_items, history_items, user_emb, item_emb,
           att_w1, att_b1, att_w2, att_b2,
           mlp_w1, mlp_b1, mlp_w2, mlp_b2, mlp_w3, mlp_b3):
    bf16 = jnp.bfloat16
    wa, wb, wc, wdm = (att_w1[:_D], att_w1[_D:2 * _D],
                       att_w1[2 * _D:3 * _D], att_w1[3 * _D:])
    wac = (wa + wc).astype(bf16)
    wd = wdm.astype(bf16)
    wbc = (wb - wc).astype(bf16)
    b1 = att_b1.reshape(1, 64)
    w2r = att_w2.reshape(1, 64)
    mw1 = mlp_w1.astype(bf16)
    mb1 = mlp_b1.reshape(1, 64)
    mw2 = mlp_w2.astype(bf16)
    mb2 = mlp_b2.reshape(1, 32)
    mw3r = mlp_w3.reshape(1, 32)
    mb3 = mlp_b3.reshape(1, 1)

    def pack(table):
        tt = jnp.swapaxes(table, 0, 1)
        t3 = tt.reshape(_D, _V // 4, 4)
        return jnp.transpose(t3, (1, 2, 0)).reshape(_V // 4, 4 * _D)

    item128 = pack(item_emb)
    user128 = pack(user_emb)

    uid = user_ids.astype(jnp.int32)
    cid = candidate_items.astype(jnp.int32)
    hid = history_items.astype(jnp.int32)

    uq = (uid & 3).reshape(_B, 1)
    iq = (cid & 3).reshape(_B, 1)

    bc = _B // _NCHUNK
    item_v, hist0 = _sc_gather2(item128, cid >> 2,
                                (hid[:bc] >> 2).reshape(-1))

    hists = [hist0.reshape(bc, _H, 128)]
    for c in range(1, _NCHUNK):
        midx = (hid[c * bc:(c + 1) * bc] >> 2).reshape(-1)
        hists.append(_sc_gather_macro(item128, midx).reshape(bc, _H, 128))

    user_v = _sc_gather_macro(user128, uid >> 2)

    parts = []
    for c in range(_NCHUNK):
        sl = slice(c * bc, (c + 1) * bc)
        parts.append(_tc_attention(
            hists[c], hid[sl], item_v[sl], iq[sl], wac, wd, wbc, b1, w2r))
    part = jnp.concatenate(parts, axis=0)
    out = _tc_final(user_v, uq, part, mw1, mb1, mw2, mb2, mw3r, mb3)
    return out[:, 0]
